# baseline (speedup 1.0000x reference)
import dataclasses
import functools

import jax
import jax.numpy as jnp
from jax import lax
from jax.experimental import pallas as pl
from jax.experimental.pallas import tpu as pltpu
from jax.experimental.pallas import tpu_sc as plsc

N = 10000
E = 320000
D = 128
NC = 2
NS = 16
CH = 128
ET = E // (NC * NS)
NCHUNK = 80
NCHUNK_G = NCHUNK
BIG = 2
SHIFT = 14
R = 10240
RPT = R // NS
TRASH = N

_MESH = plsc.VectorSubcoreMesh(
    core_axis_name="c", subcore_axis_name="s", num_cores=NC, num_subcores=NS
)

_CP = pltpu.CompilerParams()
if "needs_layout_passes" in pltpu.CompilerParams.__dataclass_fields__:
    _CP = dataclasses.replace(_CP, needs_layout_passes=False)


def _deg_body(pidx_hbm, zero_hbm, deg_hbm, dst_v, deg_v, slab_v, res_v,
              stage_sh):
    c = lax.axis_index("c")
    s = lax.axis_index("s")
    pltpu.sync_copy(zero_hbm, deg_v)
    pltpu.sync_copy(pidx_hbm.at[c].at[s], dst_v)
    ones = jnp.ones((16,), jnp.float32)
    sh = jnp.full((16,), SHIFT, jnp.int32)

    @pl.loop(0, NCHUNK_G)
    def _(j):
        @pl.loop(0, CH // 16)
        def _(k):
            idx = jax.lax.shift_right_logical(
                dst_v[j, pl.ds(k * 16, 16)], sh)
            plsc.addupdate_scatter(deg_v, [idx], ones)

    pltpu.sync_copy(deg_v, stage_sh.at[s])
    plsc.subcore_barrier()
    pltpu.sync_copy(stage_sh.at[:, pl.ds(s * RPT, RPT)], slab_v)

    @pl.loop(0, RPT // 16)
    def _(i):
        tot = jnp.zeros((16,), jnp.float32)
        for r in range(NS):
            tot = tot + slab_v[r, pl.ds(i * 16, 16)]
        res_v[pl.ds(i * 16, 16)] = tot

    pltpu.sync_copy(res_v, deg_hbm.at[pl.ds(c * R + s * RPT, RPT)])


def _deg_sc(pidx, zeroR):
    return pl.kernel(
        _deg_body,
        out_type=jax.ShapeDtypeStruct((NC * R,), jnp.float32),
        mesh=_MESH,
        scratch_types=[
            pltpu.VMEM((NCHUNK_G, CH), jnp.int32),
            pltpu.VMEM((R,), jnp.float32),
            pltpu.VMEM((NS, RPT), jnp.float32),
            pltpu.VMEM((RPT,), jnp.float32),
            pltpu.VMEM_SHARED((NS, R), jnp.float32),
        ],
        compiler_params=_CP,
    )(pidx, zeroR)


def _unpack(pidx_v, j, out1d, slot, shift):
    mask = jnp.full((16,), (1 << SHIFT) - 1, jnp.int32)
    for k in range(CH // 16):
        v = pidx_v[j, pl.ds(k * 16, 16)]
        if shift:
            v = jax.lax.shift_right_logical(v, jnp.full((16,), SHIFT, jnp.int32))
        else:
            v = jax.lax.bitwise_and(v, mask)
        out1d[pl.ds(slot * CH + k * 16, 16)] = v


def _msg_body(g_hbm, pidx_hbm, zero_hbm, part_hbm,
              pidx_v, srows, drows, rows0, acc_sh):
    c = lax.axis_index("c")
    s = lax.axis_index("s")
    pltpu.sync_copy(zero_hbm, acc_sh.at[pl.ds(s * RPT, RPT)])
    pltpu.sync_copy(pidx_hbm.at[c].at[s], pidx_v)
    plsc.subcore_barrier()

    @pl.loop(0, NCHUNK)
    def _(j):
        _unpack(pidx_v, j, srows, 0, shift=False)
        _unpack(pidx_v, j, drows, 0, shift=True)
        pltpu.sync_copy(g_hbm.at[srows], rows0)
        pltpu.sync_copy(rows0, acc_sh.at[drows], add=True)

    plsc.subcore_barrier()
    sl = pl.ds(s * RPT, RPT)
    pltpu.sync_copy(acc_sh.at[sl], part_hbm.at[c].at[sl])


def _msg_sc(g, pidx, zero):
    return pl.kernel(
        _msg_body,
        out_type=jax.ShapeDtypeStruct((NC, R, D), jnp.float32),
        mesh=_MESH,
        scratch_types=[
            pltpu.VMEM((NCHUNK_G, CH), jnp.int32),
            pltpu.VMEM((CH,), jnp.int32),
            pltpu.VMEM((CH,), jnp.int32),
            pltpu.VMEM((CH, D), jnp.float32),
            pltpu.VMEM_SHARED((R, D), jnp.float32),
        ],
        compiler_params=_CP,
    )(g, pidx, zero)


BR = 1000


def _mm_body(x_ref, w_ref, o_ref):
    o_ref[...] = jnp.dot(x_ref[...], w_ref[...],
                         preferred_element_type=jnp.float32)


def _mm_tc(x, w):
    return pl.pallas_call(
        _mm_body,
        grid=(N // BR,),
        in_specs=[
            pl.BlockSpec((BR, D), lambda i: (i, 0)),
            pl.BlockSpec((D, D), lambda i: (0, 0)),
        ],
        out_specs=pl.BlockSpec((BR, D), lambda i: (i, 0)),
        out_shape=jax.ShapeDtypeStruct((N, D), jnp.float32),
    )(x, w)


def _dinv_deg(dp):
    deg = dp[0] + dp[1] + 1.0
    return lax.rsqrt(deg), deg


def _scale_body(h_ref, dp_ref, g_ref):
    dinv, _ = _dinv_deg(dp_ref[...])
    g_ref[...] = h_ref[...] * dinv


def _scale_tc(h, degp):
    return pl.pallas_call(
        _scale_body,
        grid=(N // BR,),
        in_specs=[
            pl.BlockSpec((BR, D), lambda i: (i, 0)),
            pl.BlockSpec((NC, BR, 1), lambda i: (0, i, 0)),
        ],
        out_specs=pl.BlockSpec((BR, D), lambda i: (i, 0)),
        out_shape=jax.ShapeDtypeStruct((N, D), jnp.float32),
    )(h, degp)


def _mid_body(p_ref, h_ref, dp_ref, b_ref, w_ref, hn_ref, g_ref):
    dinv, deg = _dinv_deg(dp_ref[...])
    out = dinv * (p_ref[0] + p_ref[1]) + h_ref[...] / deg + b_ref[...]
    hn = jnp.dot(jnp.maximum(out, 0.0), w_ref[...],
                 preferred_element_type=jnp.float32)
    hn_ref[...] = hn
    g_ref[...] = hn * dinv


def _mid_tc(part, h, degp, b, w):
    return pl.pallas_call(
        _mid_body,
        grid=(N // BR,),
        in_specs=[
            pl.BlockSpec((NC, BR, D), lambda i: (0, i, 0)),
            pl.BlockSpec((BR, D), lambda i: (i, 0)),
            pl.BlockSpec((NC, BR, 1), lambda i: (0, i, 0)),
            pl.BlockSpec((1, D), lambda i: (0, 0)),
            pl.BlockSpec((D, D), lambda i: (0, 0)),
        ],
        out_specs=[
            pl.BlockSpec((BR, D), lambda i: (i, 0)),
            pl.BlockSpec((BR, D), lambda i: (i, 0)),
        ],
        out_shape=[
            jax.ShapeDtypeStruct((N, D), jnp.float32),
            jax.ShapeDtypeStruct((N, D), jnp.float32),
        ],
    )(part, h, degp, b, w)


def _fin_body(p_ref, h_ref, dp_ref, b_ref, o_ref):
    dinv, deg = _dinv_deg(dp_ref[...])
    o_ref[...] = dinv * (p_ref[0] + p_ref[1]) + h_ref[...] / deg + b_ref[...]


def _fin_tc(part, h, degp, b):
    return pl.pallas_call(
        _fin_body,
        grid=(N // BR,),
        in_specs=[
            pl.BlockSpec((NC, BR, D), lambda i: (0, i, 0)),
            pl.BlockSpec((BR, D), lambda i: (i, 0)),
            pl.BlockSpec((NC, BR, 1), lambda i: (0, i, 0)),
            pl.BlockSpec((1, D), lambda i: (0, 0)),
        ],
        out_specs=pl.BlockSpec((BR, D), lambda i: (i, 0)),
        out_shape=jax.ShapeDtypeStruct((N, D), jnp.float32),
    )(part, h, degp, b)


def kernel(x, edge_index, W0, b0, W1, b1, W2, b2):
    src = edge_index[0].reshape(NC, NS, ET)
    dst = edge_index[1].reshape(NC, NS, ET)
    pad = NCHUNK_G * CH - ET
    srcp = jnp.pad(src, ((0, 0), (0, 0), (0, pad)))
    dstp = jnp.pad(dst, ((0, 0), (0, 0), (0, pad)), constant_values=TRASH)
    pidx = ((dstp << SHIFT) | srcp).reshape(NC, NS, NCHUNK_G, CH)

    zeroR = jnp.zeros((R,), jnp.float32)
    zeroD = jnp.zeros((RPT, D), jnp.float32)

    degp = _deg_sc(pidx, zeroR).reshape(NC, R, 1)
    h = _mm_tc(x, W0)
    g = _scale_tc(h, degp)
    for b, W in ((b0, W1), (b1, W2)):
        part = _msg_sc(g, pidx, zeroD)
        h, g = _mid_tc(part, h, degp, b.reshape(1, D), W)
    part = _msg_sc(g, pidx, zeroD)
    return _fin_tc(part, h, degp, b2.reshape(1, D))


# speedup vs baseline: 1.0114x; 1.0114x over previous
import dataclasses
import functools

import jax
import jax.numpy as jnp
from jax import lax
from jax.experimental import pallas as pl
from jax.experimental.pallas import tpu as pltpu
from jax.experimental.pallas import tpu_sc as plsc

N = 10000
E = 320000
D = 128
NC = 2
NS = 16
CH = 128
ET = E // (NC * NS)
NCHUNK = 80
NCHUNK_G = NCHUNK
BIG = 2
SHIFT = 14
R = 10240
RPT = R // NS
TRASH = N

_MESH = plsc.VectorSubcoreMesh(
    core_axis_name="c", subcore_axis_name="s", num_cores=NC, num_subcores=NS
)

_CP = pltpu.CompilerParams()
if "needs_layout_passes" in pltpu.CompilerParams.__dataclass_fields__:
    _CP = dataclasses.replace(_CP, needs_layout_passes=False)


def _deg_body(dstp_hbm, zero_hbm, deg_hbm, dst_v, deg_v, slab_v, res_v,
              stage_sh):
    c = lax.axis_index("c")
    s = lax.axis_index("s")
    pltpu.sync_copy(zero_hbm, deg_v)
    pltpu.sync_copy(dstp_hbm.at[c].at[s], dst_v)
    ones = jnp.ones((16,), jnp.float32)

    @pl.loop(0, NCHUNK)
    def _(j):
        @pl.loop(0, CH // 16)
        def _(k):
            idx = dst_v[j, pl.ds(k * 16, 16)]
            plsc.addupdate_scatter(deg_v, [idx], ones)

    pltpu.sync_copy(deg_v, stage_sh.at[s])
    plsc.subcore_barrier()
    pltpu.sync_copy(stage_sh.at[:, pl.ds(s * RPT, RPT)], slab_v)

    @pl.loop(0, RPT // 16)
    def _(i):
        tot = jnp.zeros((16,), jnp.float32)
        for r in range(NS):
            tot = tot + slab_v[r, pl.ds(i * 16, 16)]
        res_v[pl.ds(i * 16, 16)] = tot

    pltpu.sync_copy(res_v, deg_hbm.at[pl.ds(c * R + s * RPT, RPT)])


def _deg_sc(dstp, zeroR):
    return pl.kernel(
        _deg_body,
        out_type=jax.ShapeDtypeStruct((NC * R,), jnp.float32),
        mesh=_MESH,
        scratch_types=[
            pltpu.VMEM((NCHUNK, CH), jnp.int32),
            pltpu.VMEM((R,), jnp.float32),
            pltpu.VMEM((NS, RPT), jnp.float32),
            pltpu.VMEM((RPT,), jnp.float32),
            pltpu.VMEM_SHARED((NS, R), jnp.float32),
        ],
        compiler_params=_CP,
    )(dstp, zeroR)


def _msg_body(g_hbm, srcp_hbm, dstp_hbm, zero_hbm, part_hbm,
              src_v, dst_v, rows_v, acc_sh):
    c = lax.axis_index("c")
    s = lax.axis_index("s")
    pltpu.sync_copy(zero_hbm, acc_sh.at[pl.ds(s * RPT, RPT)])
    pltpu.sync_copy(srcp_hbm.at[c].at[s], src_v)
    pltpu.sync_copy(dstp_hbm.at[c].at[s], dst_v)
    plsc.subcore_barrier()

    @pl.loop(0, NCHUNK)
    def _(j):
        pltpu.sync_copy(g_hbm.at[src_v.at[j]], rows_v)
        pltpu.sync_copy(rows_v, acc_sh.at[dst_v.at[j]], add=True)

    plsc.subcore_barrier()
    sl = pl.ds(s * RPT, RPT)
    pltpu.sync_copy(acc_sh.at[sl], part_hbm.at[c].at[sl])


def _msg_sc(g, srcp, dstp, zero):
    return pl.kernel(
        _msg_body,
        out_type=jax.ShapeDtypeStruct((NC, R, D), jnp.float32),
        mesh=_MESH,
        scratch_types=[
            pltpu.VMEM((NCHUNK, CH), jnp.int32),
            pltpu.VMEM((NCHUNK, CH), jnp.int32),
            pltpu.VMEM((CH, D), jnp.float32),
            pltpu.VMEM_SHARED((R, D), jnp.float32),
        ],
    )(g, srcp, dstp, zero)


BR = 1000


def _mm_body(x_ref, w_ref, o_ref):
    o_ref[...] = jnp.dot(x_ref[...], w_ref[...],
                         preferred_element_type=jnp.float32)


def _mm_tc(x, w):
    return pl.pallas_call(
        _mm_body,
        grid=(N // BR,),
        in_specs=[
            pl.BlockSpec((BR, D), lambda i: (i, 0)),
            pl.BlockSpec((D, D), lambda i: (0, 0)),
        ],
        out_specs=pl.BlockSpec((BR, D), lambda i: (i, 0)),
        out_shape=jax.ShapeDtypeStruct((N, D), jnp.float32),
    )(x, w)


def _dinv_deg(dp):
    deg = dp[0] + dp[1] + 1.0
    return lax.rsqrt(deg), deg


def _scale_body(h_ref, dp_ref, g_ref):
    dinv, _ = _dinv_deg(dp_ref[...])
    g_ref[...] = h_ref[...] * dinv


def _scale_tc(h, degp):
    return pl.pallas_call(
        _scale_body,
        grid=(N // BR,),
        in_specs=[
            pl.BlockSpec((BR, D), lambda i: (i, 0)),
            pl.BlockSpec((NC, BR, 1), lambda i: (0, i, 0)),
        ],
        out_specs=pl.BlockSpec((BR, D), lambda i: (i, 0)),
        out_shape=jax.ShapeDtypeStruct((N, D), jnp.float32),
    )(h, degp)


def _mid_body(p_ref, h_ref, dp_ref, b_ref, w_ref, hn_ref, g_ref):
    dinv, deg = _dinv_deg(dp_ref[...])
    out = dinv * (p_ref[0] + p_ref[1]) + h_ref[...] / deg + b_ref[...]
    hn = jnp.dot(jnp.maximum(out, 0.0), w_ref[...],
                 preferred_element_type=jnp.float32)
    hn_ref[...] = hn
    g_ref[...] = hn * dinv


def _mid_tc(part, h, degp, b, w):
    return pl.pallas_call(
        _mid_body,
        grid=(N // BR,),
        in_specs=[
            pl.BlockSpec((NC, BR, D), lambda i: (0, i, 0)),
            pl.BlockSpec((BR, D), lambda i: (i, 0)),
            pl.BlockSpec((NC, BR, 1), lambda i: (0, i, 0)),
            pl.BlockSpec((1, D), lambda i: (0, 0)),
            pl.BlockSpec((D, D), lambda i: (0, 0)),
        ],
        out_specs=[
            pl.BlockSpec((BR, D), lambda i: (i, 0)),
            pl.BlockSpec((BR, D), lambda i: (i, 0)),
        ],
        out_shape=[
            jax.ShapeDtypeStruct((N, D), jnp.float32),
            jax.ShapeDtypeStruct((N, D), jnp.float32),
        ],
    )(part, h, degp, b, w)


def _fin_body(p_ref, h_ref, dp_ref, b_ref, o_ref):
    dinv, deg = _dinv_deg(dp_ref[...])
    o_ref[...] = dinv * (p_ref[0] + p_ref[1]) + h_ref[...] / deg + b_ref[...]


def _fin_tc(part, h, degp, b):
    return pl.pallas_call(
        _fin_body,
        grid=(N // BR,),
        in_specs=[
            pl.BlockSpec((NC, BR, D), lambda i: (0, i, 0)),
            pl.BlockSpec((BR, D), lambda i: (i, 0)),
            pl.BlockSpec((NC, BR, 1), lambda i: (0, i, 0)),
            pl.BlockSpec((1, D), lambda i: (0, 0)),
        ],
        out_specs=pl.BlockSpec((BR, D), lambda i: (i, 0)),
        out_shape=jax.ShapeDtypeStruct((N, D), jnp.float32),
    )(part, h, degp, b)


def kernel(x, edge_index, W0, b0, W1, b1, W2, b2):
    src = edge_index[0].reshape(NC, NS, ET)
    dst = edge_index[1].reshape(NC, NS, ET)
    pad = NCHUNK * CH - ET
    srcp = jnp.pad(src, ((0, 0), (0, 0), (0, pad))).reshape(NC, NS, NCHUNK, CH)
    dstp = jnp.pad(dst, ((0, 0), (0, 0), (0, pad)),
                   constant_values=TRASH).reshape(NC, NS, NCHUNK, CH)

    zeroR = jnp.zeros((R,), jnp.float32)
    zeroD = jnp.zeros((RPT, D), jnp.float32)

    degp = _deg_sc(dstp, zeroR).reshape(NC, R, 1)
    h = _mm_tc(x, W0)
    g = _scale_tc(h, degp)
    for b, W in ((b0, W1), (b1, W2)):
        part = _msg_sc(g, srcp, dstp, zeroD)
        h, g = _mid_tc(part, h, degp, b.reshape(1, D), W)
    part = _msg_sc(g, srcp, dstp, zeroD)
    return _fin_tc(part, h, degp, b2.reshape(1, D))


# speedup vs baseline: 1.4690x; 1.4525x over previous
import dataclasses
import functools

import jax
import jax.numpy as jnp
from jax import lax
from jax.experimental import pallas as pl
from jax.experimental.pallas import tpu as pltpu
from jax.experimental.pallas import tpu_sc as plsc

N = 10000
E = 320000
D = 128
NC = 2
NS = 16
CH = 128
ET = E // (NC * NS)
NCHUNK = -(-ET // CH)
R = 10240
RPT = R // NS
TRASH = N

_MESH = plsc.VectorSubcoreMesh(
    core_axis_name="c", subcore_axis_name="s", num_cores=NC, num_subcores=NS
)

_CP = pltpu.CompilerParams()
if "needs_layout_passes" in pltpu.CompilerParams.__dataclass_fields__:
    _CP = dataclasses.replace(_CP, needs_layout_passes=False)


def _deg_body(dstp_hbm, zero_hbm, deg_hbm, dst_v, deg_v, slab_v, res_v,
              stage_sh):
    c = lax.axis_index("c")
    s = lax.axis_index("s")
    pltpu.sync_copy(zero_hbm, deg_v)
    pltpu.sync_copy(dstp_hbm.at[c].at[s], dst_v)
    ones = jnp.ones((16,), jnp.float32)

    @pl.loop(0, NCHUNK)
    def _(j):
        @pl.loop(0, CH // 16)
        def _(k):
            idx = dst_v[j, pl.ds(k * 16, 16)]
            plsc.addupdate_scatter(deg_v, [idx], ones)

    pltpu.sync_copy(deg_v, stage_sh.at[s])
    plsc.subcore_barrier()
    pltpu.sync_copy(stage_sh.at[:, pl.ds(s * RPT, RPT)], slab_v)

    @pl.loop(0, RPT // 16)
    def _(i):
        tot = jnp.zeros((16,), jnp.float32)
        for r in range(NS):
            tot = tot + slab_v[r, pl.ds(i * 16, 16)]
        res_v[pl.ds(i * 16, 16)] = tot

    pltpu.sync_copy(res_v, deg_hbm.at[pl.ds(c * R + s * RPT, RPT)])


def _deg_sc(dstp, zeroR):
    return pl.kernel(
        _deg_body,
        out_type=jax.ShapeDtypeStruct((NC * R,), jnp.float32),
        mesh=_MESH,
        scratch_types=[
            pltpu.VMEM((NCHUNK, CH), jnp.int32),
            pltpu.VMEM((R,), jnp.float32),
            pltpu.VMEM((NS, RPT), jnp.float32),
            pltpu.VMEM((RPT,), jnp.float32),
            pltpu.VMEM_SHARED((NS, R), jnp.float32),
        ],
        compiler_params=_CP,
    )(dstp, zeroR)


def _msg_body(g_hbm, srcp_hbm, dstp_hbm, zero_hbm, part_hbm,
              src_v, dst_v, rows_v, acc_sh):
    c = lax.axis_index("c")
    s = lax.axis_index("s")
    pltpu.sync_copy(zero_hbm, acc_sh.at[pl.ds(s * RPT, RPT)])
    pltpu.sync_copy(srcp_hbm.at[c].at[s], src_v)
    pltpu.sync_copy(dstp_hbm.at[c].at[s], dst_v)
    plsc.subcore_barrier()

    @pl.loop(0, NCHUNK)
    def _(j):
        pltpu.sync_copy(g_hbm.at[src_v.at[j]], rows_v)
        pltpu.sync_copy(rows_v, acc_sh.at[dst_v.at[j]], add=True)

    plsc.subcore_barrier()
    sl = pl.ds(s * RPT, RPT)
    pltpu.sync_copy(acc_sh.at[sl], part_hbm.at[c].at[sl])


def _msg_sc(g, srcp, dstp, zero):
    return pl.kernel(
        _msg_body,
        out_type=jax.ShapeDtypeStruct((NC, R, D), jnp.float32),
        mesh=_MESH,
        scratch_types=[
            pltpu.VMEM((NCHUNK, CH), jnp.int32),
            pltpu.VMEM((NCHUNK, CH), jnp.int32),
            pltpu.VMEM((CH, D), jnp.float32),
            pltpu.VMEM_SHARED((R, D), jnp.float32),
        ],
    )(g, srcp, dstp, zero)


BR = 1000


def _mm_body(x_ref, w_ref, o_ref):
    o_ref[...] = jnp.dot(x_ref[...], w_ref[...],
                         preferred_element_type=jnp.float32)


def _mm_tc(x, w):
    return pl.pallas_call(
        _mm_body,
        grid=(N // BR,),
        in_specs=[
            pl.BlockSpec((BR, D), lambda i: (i, 0)),
            pl.BlockSpec((D, D), lambda i: (0, 0)),
        ],
        out_specs=pl.BlockSpec((BR, D), lambda i: (i, 0)),
        out_shape=jax.ShapeDtypeStruct((N, D), jnp.float32),
    )(x, w)


def _dinv_deg(dp):
    deg = dp[0] + dp[1] + 1.0
    return lax.rsqrt(deg), deg


def _scale_body(h_ref, dp_ref, g_ref):
    dinv, _ = _dinv_deg(dp_ref[...])
    g_ref[...] = h_ref[...] * dinv


def _scale_tc(h, degp):
    return pl.pallas_call(
        _scale_body,
        grid=(N // BR,),
        in_specs=[
            pl.BlockSpec((BR, D), lambda i: (i, 0)),
            pl.BlockSpec((NC, BR, 1), lambda i: (0, i, 0)),
        ],
        out_specs=pl.BlockSpec((BR, D), lambda i: (i, 0)),
        out_shape=jax.ShapeDtypeStruct((N, D), jnp.float32),
    )(h, degp)


def _mid_body(p_ref, h_ref, dp_ref, b_ref, w_ref, hn_ref, g_ref):
    dinv, deg = _dinv_deg(dp_ref[...])
    out = dinv * (p_ref[0] + p_ref[1]) + h_ref[...] / deg + b_ref[...]
    hn = jnp.dot(jnp.maximum(out, 0.0), w_ref[...],
                 preferred_element_type=jnp.float32)
    hn_ref[...] = hn
    g_ref[...] = hn * dinv


def _mid_tc(part, h, degp, b, w):
    return pl.pallas_call(
        _mid_body,
        grid=(N // BR,),
        in_specs=[
            pl.BlockSpec((NC, BR, D), lambda i: (0, i, 0)),
            pl.BlockSpec((BR, D), lambda i: (i, 0)),
            pl.BlockSpec((NC, BR, 1), lambda i: (0, i, 0)),
            pl.BlockSpec((1, D), lambda i: (0, 0)),
            pl.BlockSpec((D, D), lambda i: (0, 0)),
        ],
        out_specs=[
            pl.BlockSpec((BR, D), lambda i: (i, 0)),
            pl.BlockSpec((BR, D), lambda i: (i, 0)),
        ],
        out_shape=[
            jax.ShapeDtypeStruct((N, D), jnp.float32),
            jax.ShapeDtypeStruct((N, D), jnp.float32),
        ],
    )(part, h, degp, b, w)


def _fin_body(p_ref, h_ref, dp_ref, b_ref, o_ref):
    dinv, deg = _dinv_deg(dp_ref[...])
    o_ref[...] = dinv * (p_ref[0] + p_ref[1]) + h_ref[...] / deg + b_ref[...]


def _fin_tc(part, h, degp, b):
    return pl.pallas_call(
        _fin_body,
        grid=(N // BR,),
        in_specs=[
            pl.BlockSpec((NC, BR, D), lambda i: (0, i, 0)),
            pl.BlockSpec((BR, D), lambda i: (i, 0)),
            pl.BlockSpec((NC, BR, 1), lambda i: (0, i, 0)),
            pl.BlockSpec((1, D), lambda i: (0, 0)),
        ],
        out_specs=pl.BlockSpec((BR, D), lambda i: (i, 0)),
        out_shape=jax.ShapeDtypeStruct((N, D), jnp.float32),
    )(part, h, degp, b)


def kernel(x, edge_index, W0, b0, W1, b1, W2, b2):
    src = edge_index[0].reshape(NC, NS, ET)
    dst = edge_index[1].reshape(NC, NS, ET)
    pad = NCHUNK * CH - ET
    srcp = jnp.pad(src, ((0, 0), (0, 0), (0, pad))).reshape(NC, NS, NCHUNK, CH)
    trash = TRASH + (jnp.arange(pad, dtype=jnp.int32) % (R - N))
    dstp = jnp.concatenate(
        [dst, jnp.broadcast_to(trash, (NC, NS, pad))], axis=2
    ).reshape(NC, NS, NCHUNK, CH)

    zeroR = jnp.zeros((R,), jnp.float32)
    zeroD = jnp.zeros((RPT, D), jnp.float32)

    degp = _deg_sc(dstp, zeroR).reshape(NC, R, 1)
    h = _mm_tc(x, W0)
    g = _scale_tc(h, degp)
    for b, W in ((b0, W1), (b1, W2)):
        part = _msg_sc(g, srcp, dstp, zeroD)
        h, g = _mid_tc(part, h, degp, b.reshape(1, D), W)
    part = _msg_sc(g, srcp, dstp, zeroD)
    return _fin_tc(part, h, degp, b2.reshape(1, D))


# speedup vs baseline: 1.4697x; 1.0004x over previous
import dataclasses

import jax
import jax.numpy as jnp
from jax import lax
from jax.experimental import pallas as pl
from jax.experimental.pallas import tpu as pltpu
from jax.experimental.pallas import tpu_sc as plsc

N = 10000
E = 320000
D = 128
NC = 2
NS = 16
CH = 128
ET = E // (NC * NS)
NCHUNK = -(-ET // CH)
R = 10240
RPT = R // NS
TRASH = N

_MESH = plsc.VectorSubcoreMesh(
    core_axis_name="c", subcore_axis_name="s", num_cores=NC, num_subcores=NS
)

_CP = pltpu.CompilerParams()
if "needs_layout_passes" in pltpu.CompilerParams.__dataclass_fields__:
    _CP = dataclasses.replace(_CP, needs_layout_passes=False)


def _deg_body(dstp_hbm, zero_hbm, deg_hbm, dst_v, deg_v, slab_v, res_v,
              stage_sh):
    c = lax.axis_index("c")
    s = lax.axis_index("s")
    pltpu.sync_copy(zero_hbm, deg_v)
    pltpu.sync_copy(dstp_hbm.at[c].at[s], dst_v)
    ones = jnp.ones((16,), jnp.float32)

    @pl.loop(0, NCHUNK)
    def _(j):
        @pl.loop(0, CH // 16)
        def _(k):
            idx = dst_v[j, pl.ds(k * 16, 16)]
            plsc.addupdate_scatter(deg_v, [idx], ones)

    pltpu.sync_copy(deg_v, stage_sh.at[s])
    plsc.subcore_barrier()
    pltpu.sync_copy(stage_sh.at[:, pl.ds(s * RPT, RPT)], slab_v)

    @pl.loop(0, RPT // 16)
    def _(i):
        tot = jnp.zeros((16,), jnp.float32)
        for r in range(NS):
            tot = tot + slab_v[r, pl.ds(i * 16, 16)]
        res_v[pl.ds(i * 16, 16)] = tot

    pltpu.sync_copy(res_v, deg_hbm.at[pl.ds(c * R + s * RPT, RPT)])


def _deg_sc(dstp, zeroR):
    return pl.kernel(
        _deg_body,
        out_type=jax.ShapeDtypeStruct((NC * R,), jnp.float32),
        mesh=_MESH,
        scratch_types=[
            pltpu.VMEM((NCHUNK, CH), jnp.int32),
            pltpu.VMEM((R,), jnp.float32),
            pltpu.VMEM((NS, RPT), jnp.float32),
            pltpu.VMEM((RPT,), jnp.float32),
            pltpu.VMEM_SHARED((NS, R), jnp.float32),
        ],
        compiler_params=_CP,
    )(dstp, zeroR)


def _msg_body(g_hbm, srcp_hbm, dstp_hbm, zero_hbm, part_hbm,
              src_v, dst_v, rows_v, acc_sh):
    c = lax.axis_index("c")
    s = lax.axis_index("s")
    pltpu.sync_copy(zero_hbm, acc_sh.at[pl.ds(s * RPT, RPT)])
    pltpu.sync_copy(srcp_hbm.at[c].at[s], src_v)
    pltpu.sync_copy(dstp_hbm.at[c].at[s], dst_v)
    plsc.subcore_barrier()

    @pl.loop(0, NCHUNK)
    def _(j):
        pltpu.sync_copy(g_hbm.at[src_v.at[j]], rows_v)
        pltpu.sync_copy(rows_v, acc_sh.at[dst_v.at[j]], add=True)

    plsc.subcore_barrier()
    sl = pl.ds(s * RPT, RPT)
    pltpu.sync_copy(acc_sh.at[sl], part_hbm.at[c].at[sl])


def _msg_sc(g, srcp, dstp, zero):
    return pl.kernel(
        _msg_body,
        out_type=jax.ShapeDtypeStruct((NC, R, D), jnp.float32),
        mesh=_MESH,
        scratch_types=[
            pltpu.VMEM((NCHUNK, CH), jnp.int32),
            pltpu.VMEM((NCHUNK, CH), jnp.int32),
            pltpu.VMEM((CH, D), jnp.float32),
            pltpu.VMEM_SHARED((R, D), jnp.float32),
        ],
    )(g, srcp, dstp, zero)


BR = 1000


def _mm_body(x_ref, w_ref, o_ref):
    o_ref[...] = jnp.dot(x_ref[...], w_ref[...],
                         preferred_element_type=jnp.float32)


def _mm_tc(x, w):
    return pl.pallas_call(
        _mm_body,
        grid=(N // BR,),
        in_specs=[
            pl.BlockSpec((BR, D), lambda i: (i, 0)),
            pl.BlockSpec((D, D), lambda i: (0, 0)),
        ],
        out_specs=pl.BlockSpec((BR, D), lambda i: (i, 0)),
        out_shape=jax.ShapeDtypeStruct((N, D), jnp.float32),
    )(x, w)


def _dinv_deg(dp):
    deg = dp[0] + dp[1] + 1.0
    return lax.rsqrt(deg), deg


def _scale_body(h_ref, dp_ref, g_ref):
    dinv, _ = _dinv_deg(dp_ref[...])
    g_ref[...] = h_ref[...] * dinv


def _scale_tc(h, degp):
    return pl.pallas_call(
        _scale_body,
        grid=(N // BR,),
        in_specs=[
            pl.BlockSpec((BR, D), lambda i: (i, 0)),
            pl.BlockSpec((NC, BR, 1), lambda i: (0, i, 0)),
        ],
        out_specs=pl.BlockSpec((BR, D), lambda i: (i, 0)),
        out_shape=jax.ShapeDtypeStruct((N, D), jnp.float32),
    )(h, degp)


def _mid_body(p_ref, h_ref, dp_ref, b_ref, w_ref, hn_ref, g_ref):
    dinv, deg = _dinv_deg(dp_ref[...])
    out = dinv * (p_ref[0] + p_ref[1]) + h_ref[...] / deg + b_ref[...]
    hn = jnp.dot(jnp.maximum(out, 0.0), w_ref[...],
                 preferred_element_type=jnp.float32)
    hn_ref[...] = hn
    g_ref[...] = hn * dinv


def _mid_tc(part, h, degp, b, w):
    return pl.pallas_call(
        _mid_body,
        grid=(N // BR,),
        in_specs=[
            pl.BlockSpec((NC, BR, D), lambda i: (0, i, 0)),
            pl.BlockSpec((BR, D), lambda i: (i, 0)),
            pl.BlockSpec((NC, BR, 1), lambda i: (0, i, 0)),
            pl.BlockSpec((1, D), lambda i: (0, 0)),
            pl.BlockSpec((D, D), lambda i: (0, 0)),
        ],
        out_specs=[
            pl.BlockSpec((BR, D), lambda i: (i, 0)),
            pl.BlockSpec((BR, D), lambda i: (i, 0)),
        ],
        out_shape=[
            jax.ShapeDtypeStruct((N, D), jnp.float32),
            jax.ShapeDtypeStruct((N, D), jnp.float32),
        ],
    )(part, h, degp, b, w)


def _fin_body(p_ref, h_ref, dp_ref, b_ref, o_ref):
    dinv, deg = _dinv_deg(dp_ref[...])
    o_ref[...] = dinv * (p_ref[0] + p_ref[1]) + h_ref[...] / deg + b_ref[...]


def _fin_tc(part, h, degp, b):
    return pl.pallas_call(
        _fin_body,
        grid=(N // BR,),
        in_specs=[
            pl.BlockSpec((NC, BR, D), lambda i: (0, i, 0)),
            pl.BlockSpec((BR, D), lambda i: (i, 0)),
            pl.BlockSpec((NC, BR, 1), lambda i: (0, i, 0)),
            pl.BlockSpec((1, D), lambda i: (0, 0)),
        ],
        out_specs=pl.BlockSpec((BR, D), lambda i: (i, 0)),
        out_shape=jax.ShapeDtypeStruct((N, D), jnp.float32),
    )(part, h, degp, b)


def kernel(x, edge_index, W0, b0, W1, b1, W2, b2):
    src = edge_index[0].reshape(NC, NS, ET)
    dst = edge_index[1].reshape(NC, NS, ET)
    pad = NCHUNK * CH - ET
    srcp = jnp.pad(src, ((0, 0), (0, 0), (0, pad))).reshape(NC, NS, NCHUNK, CH)
    trash = TRASH + (jnp.arange(pad, dtype=jnp.int32) % (R - N))
    dstp = jnp.concatenate(
        [dst, jnp.broadcast_to(trash, (NC, NS, pad))], axis=2
    ).reshape(NC, NS, NCHUNK, CH)

    zeroR = jnp.zeros((R,), jnp.float32)
    zeroD = jnp.zeros((RPT, D), jnp.float32)

    degp = _deg_sc(dstp, zeroR).reshape(NC, R, 1)
    h = _mm_tc(x, W0)
    g = _scale_tc(h, degp)
    for b, W in ((b0, W1), (b1, W2)):
        part = _msg_sc(g, srcp, dstp, zeroD)
        h, g = _mid_tc(part, h, degp, b.reshape(1, D), W)
    part = _msg_sc(g, srcp, dstp, zeroD)
    return _fin_tc(part, h, degp, b2.reshape(1, D))
